# baseline (speedup 1.0000x reference)
import jax
import jax.numpy as jnp
from jax.experimental import pallas as pl
from jax.experimental.pallas import tpu as pltpu

H = 8
D = 64
M = 8192
B = 1024
K = 64
IN = 512

_TR = 4096
_NT = (H * M) // _TR
_RPH = M // _TR


def _mask_body(x_ref, wq_ref, bq_ref, bp_ref, maskt_ref):
    x = x_ref[0]
    q = jnp.sum(wq_ref[:] * x[None, None, :], axis=2) + bq_ref[:]
    qn = q * jax.lax.rsqrt(jnp.sum(q * q, axis=1, keepdims=True))
    s = jnp.sum(bp_ref[:] * qn[:, None, :], axis=2)

    i = jax.lax.bitcast_convert_type(s, jnp.int32)
    key = jnp.where(i < 0, i ^ jnp.int32(0x7FFFFFFF), i)

    def step(it, cur):
        bit = 31 - it
        cand = cur + (jnp.int32(1) << bit)
        cnt = jnp.sum((key >= cand).astype(jnp.float32), axis=1, keepdims=True)
        return jnp.where(cnt >= K, cand, cur)

    kth = jax.lax.fori_loop(0, 32, step, jnp.full((H, 1), jnp.int32(-(2**31))))

    gt = (key > kth).astype(jnp.float32)
    n_gt = jnp.sum(gt, axis=1, keepdims=True)
    tie = (key == kth).astype(jnp.float32)
    r = jax.lax.broadcasted_iota(jnp.int32, (B, B), 0)
    c = jax.lax.broadcasted_iota(jnp.int32, (B, B), 1)
    lt = (r < c).astype(jnp.float32)
    tie_rank = jnp.dot(tie, lt, preferred_element_type=jnp.float32)
    sel_tie = tie * (tie_rank < (K - n_gt)).astype(jnp.float32)
    maskt_ref[:] = (gt + sel_tie).T


def _attn_body(maskt_ref, w_ref, attn_ref):
    t = pl.program_id(0)
    h = t // _RPH
    r = jnp.dot(w_ref[:], maskt_ref[:], preferred_element_type=jnp.float32)
    col = jax.lax.broadcasted_iota(jnp.int32, (1, H), 1)
    sel = (col == h).astype(jnp.float32)
    attn_ref[0, 0, :] = jnp.sum(r * sel, axis=1) * (1.0 / K)


def _retrieve_body(attn_ref, mem_ref, out_ref):
    a = attn_ref[0]
    s = (2.0 * a) / (1.0 + a)
    l = s * 10.0
    l = l - jnp.max(l, axis=1, keepdims=True)
    e = jnp.exp(l)
    p = e / jnp.sum(e, axis=1, keepdims=True)
    mem = mem_ref[0]
    inv = jax.lax.rsqrt(jnp.sum(mem * mem, axis=1, keepdims=True))
    memn = mem * inv
    out_ref[0] = jnp.dot(p, memn, preferred_element_type=jnp.float32) * jnp.sqrt(
        float(D)
    )


def kernel(x, Wq, bq, bin_proj, weight_matrix, memories):
    x2 = x.reshape(1, IN)
    wq3 = Wq.reshape(H, D, IN)
    bq2 = bq.reshape(H, D)
    wflat = weight_matrix.reshape(H * M, B)

    maskt = pl.pallas_call(
        _mask_body,
        out_shape=jax.ShapeDtypeStruct((B, H), jnp.float32),
    )(x2, wq3, bq2, bin_proj)

    attn = pl.pallas_call(
        _attn_body,
        grid=(_NT,),
        in_specs=[
            pl.BlockSpec((B, H), lambda t: (0, 0)),
            pl.BlockSpec((_TR, B), lambda t: (t, 0)),
        ],
        out_specs=pl.BlockSpec((1, 1, _TR), lambda t: (t, 0, 0)),
        out_shape=jax.ShapeDtypeStruct((_NT, 1, _TR), jnp.float32),
    )(maskt, wflat)

    attn3 = attn.reshape(H, 1, M)

    out = pl.pallas_call(
        _retrieve_body,
        grid=(H,),
        in_specs=[
            pl.BlockSpec((1, 1, M), lambda t: (t, 0, 0)),
            pl.BlockSpec((1, M, D), lambda t: (t, 0, 0)),
        ],
        out_specs=pl.BlockSpec((1, 1, D), lambda t: (t, 0, 0)),
        out_shape=jax.ShapeDtypeStruct((H, 1, D), jnp.float32),
    )(attn3, memories)

    return out.reshape(H * D)


# speedup vs baseline: 1.0242x; 1.0242x over previous
import jax
import jax.numpy as jnp
from jax.experimental import pallas as pl
from jax.experimental.pallas import tpu as pltpu

H = 8
D = 64
M = 8192
B = 1024
K = 64
IN = 512

_TR = 2048
_NT = (H * M) // _TR
_RPH = M // _TR


def _mask_body(x_ref, wq_ref, bq_ref, bp_ref, maskt_ref):
    x = x_ref[0]
    q = jnp.sum(wq_ref[:] * x[None, None, :], axis=2) + bq_ref[:]
    qn = q * jax.lax.rsqrt(jnp.sum(q * q, axis=1, keepdims=True))
    s = jnp.sum(bp_ref[:] * qn[:, None, :], axis=2)

    i = jax.lax.bitcast_convert_type(s, jnp.int32)
    key = jnp.where(i < 0, i ^ jnp.int32(0x7FFFFFFF), i)

    def step(it, cur):
        bit = 31 - it
        cand = cur + (jnp.int32(1) << bit)
        cnt = jnp.sum((key >= cand).astype(jnp.float32), axis=1, keepdims=True)
        return jnp.where(cnt >= K, cand, cur)

    kth = jax.lax.fori_loop(0, 32, step, jnp.full((H, 1), jnp.int32(-(2**31))))

    gt = (key > kth).astype(jnp.float32)
    n_gt = jnp.sum(gt, axis=1, keepdims=True)
    tie = (key == kth).astype(jnp.float32)
    r = jax.lax.broadcasted_iota(jnp.int32, (B, B), 0)
    c = jax.lax.broadcasted_iota(jnp.int32, (B, B), 1)
    lt = (r < c).astype(jnp.float32)
    tie_rank = jnp.dot(tie, lt, preferred_element_type=jnp.float32)
    sel_tie = tie * (tie_rank < (K - n_gt)).astype(jnp.float32)
    maskt_ref[:] = (gt + sel_tie).T


def _attn_body(maskt_ref, w_ref, attn_ref):
    t = pl.program_id(0)
    h = t // _RPH
    r = jnp.dot(
        w_ref[:].astype(jnp.bfloat16),
        maskt_ref[:].astype(jnp.bfloat16),
        preferred_element_type=jnp.float32,
    )
    col = jax.lax.broadcasted_iota(jnp.int32, (1, H), 1)
    sel = (col == h).astype(jnp.float32)
    attn_ref[0, 0, :] = jnp.sum(r * sel, axis=1) * (1.0 / K)


def _retrieve_body(attn_ref, mem_ref, out_ref):
    a = attn_ref[0]
    s = (2.0 * a) / (1.0 + a)
    l = s * 10.0
    l = l - jnp.max(l, axis=1, keepdims=True)
    e = jnp.exp(l)
    p = e / jnp.sum(e, axis=1, keepdims=True)
    mem = mem_ref[0]
    inv = jax.lax.rsqrt(jnp.sum(mem * mem, axis=1, keepdims=True))
    memn = mem * inv
    out_ref[0] = jnp.dot(p, memn, preferred_element_type=jnp.float32) * jnp.sqrt(
        float(D)
    )


def kernel(x, Wq, bq, bin_proj, weight_matrix, memories):
    x2 = x.reshape(1, IN)
    wq3 = Wq.reshape(H, D, IN)
    bq2 = bq.reshape(H, D)
    wflat = weight_matrix.reshape(H * M, B)

    maskt = pl.pallas_call(
        _mask_body,
        out_shape=jax.ShapeDtypeStruct((B, H), jnp.float32),
    )(x2, wq3, bq2, bin_proj)

    attn = pl.pallas_call(
        _attn_body,
        grid=(_NT,),
        in_specs=[
            pl.BlockSpec((B, H), lambda t: (0, 0)),
            pl.BlockSpec((_TR, B), lambda t: (t, 0)),
        ],
        out_specs=pl.BlockSpec((1, 1, _TR), lambda t: (t, 0, 0)),
        out_shape=jax.ShapeDtypeStruct((_NT, 1, _TR), jnp.float32),
    )(maskt, wflat)

    attn3 = attn.reshape(H, 1, M)

    out = pl.pallas_call(
        _retrieve_body,
        grid=(H,),
        in_specs=[
            pl.BlockSpec((1, 1, M), lambda t: (t, 0, 0)),
            pl.BlockSpec((1, M, D), lambda t: (t, 0, 0)),
        ],
        out_specs=pl.BlockSpec((1, 1, D), lambda t: (t, 0, 0)),
        out_shape=jax.ShapeDtypeStruct((H, 1, D), jnp.float32),
    )(attn3, memories)

    return out.reshape(H * D)


# speedup vs baseline: 1.2354x; 1.2062x over previous
import jax
import jax.numpy as jnp
from jax.experimental import pallas as pl
from jax.experimental.pallas import tpu as pltpu

H = 8
D = 64
M = 8192
B = 1024
K = 64
IN = 512

_TR = 2048
_NT = (H * M) // _TR
_RPH = M // _TR

_NEG = -1e30


def _mask_body(x_ref, wq_ref, bq_ref, bp_ref, maskt_ref):
    x = x_ref[0]
    q = jnp.sum(wq_ref[:] * x[None, None, :], axis=2) + bq_ref[:]
    qn = q * jax.lax.rsqrt(jnp.sum(q * q, axis=1, keepdims=True))
    s = jnp.sum(bp_ref[:] * qn[:, None, :], axis=2)

    i = jax.lax.bitcast_convert_type(s, jnp.int32)
    key = jnp.where(i < 0, i ^ jnp.int32(0x7FFFFFFF), i)

    ones = jnp.ones((B, 8), jnp.bfloat16)

    def count_ge(c):
        ge = (key >= c).astype(jnp.bfloat16)
        return jnp.dot(ge, ones, preferred_element_type=jnp.float32)[:, 0:1]

    def step(it, cur):
        bit = 31 - it
        cand = cur + (jnp.int32(1) << bit)
        return jnp.where(count_ge(cand) >= K, cand, cur)

    kth = jax.lax.fori_loop(0, 32, step, jnp.full((H, 1), jnp.int32(-(2**31))))

    gt = (key > kth).astype(jnp.float32)
    n_gt = jnp.dot(gt.astype(jnp.bfloat16), ones, preferred_element_type=jnp.float32)[
        :, 0:1
    ]
    tie = (key == kth).astype(jnp.bfloat16)
    r = jax.lax.broadcasted_iota(jnp.int32, (B, B), 0)
    c = jax.lax.broadcasted_iota(jnp.int32, (B, B), 1)
    lt = (r < c).astype(jnp.bfloat16)
    tie_rank = jnp.dot(tie, lt, preferred_element_type=jnp.float32)
    sel_tie = tie.astype(jnp.float32) * (tie_rank < (K - n_gt)).astype(jnp.float32)
    maskt_ref[:] = (gt + sel_tie).T


def _flash_body(maskt_ref, w_ref, mem_ref, out_ref, stat_ref, acc_ref):
    t = pl.program_id(0)
    h = t // _RPH
    oh = (jax.lax.broadcasted_iota(jnp.int32, (1, H), 1) == h).astype(jnp.float32)
    mh = (maskt_ref[:] * oh).astype(jnp.bfloat16)
    r = jnp.dot(
        w_ref[:].astype(jnp.bfloat16), mh, preferred_element_type=jnp.float32
    )
    a = r * (1.0 / K)
    sq = (2.0 * a) / (1.0 + a) * 10.0
    l = jnp.where(oh > 0.0, sq, _NEG)
    mloc = jnp.max(jnp.max(l, axis=0, keepdims=True), axis=1, keepdims=True)

    @pl.when(t % _RPH == 0)
    def _():
        stat_ref[0:1, 0:1] = jnp.full((1, 1), _NEG, jnp.float32)
        stat_ref[1:2, 0:1] = jnp.zeros((1, 1), jnp.float32)
        acc_ref[:] = jnp.zeros_like(acc_ref)

    m_old = stat_ref[0:1, 0:1]
    s_old = stat_ref[1:2, 0:1]
    m_new = jnp.maximum(m_old, mloc)
    corr = jnp.exp(m_old - m_new)
    e = jnp.exp(l - m_new)

    mem = mem_ref[:]
    inv = jax.lax.rsqrt(jnp.sum(mem * mem, axis=1, keepdims=True))
    ew = e * inv
    contrib = jax.lax.dot_general(
        ew, mem, (((0,), (0,)), ((), ())), preferred_element_type=jnp.float32
    )
    s_loc = jnp.sum(jnp.sum(e, axis=0, keepdims=True), axis=1, keepdims=True)
    stat_ref[0:1, 0:1] = m_new
    stat_ref[1:2, 0:1] = s_old * corr + s_loc
    acc_ref[:] = acc_ref[:] * corr + contrib

    @pl.when(t % _RPH == _RPH - 1)
    def _():
        s_tot = stat_ref[1:2, 0:1]
        o = jnp.sum(acc_ref[:], axis=0, keepdims=True) / s_tot * jnp.sqrt(float(D))
        out_ref[0] = o


def kernel(x, Wq, bq, bin_proj, weight_matrix, memories):
    x2 = x.reshape(1, IN)
    wq3 = Wq.reshape(H, D, IN)
    bq2 = bq.reshape(H, D)
    wflat = weight_matrix.reshape(H * M, B)
    memflat = memories.reshape(H * M, D)

    maskt = pl.pallas_call(
        _mask_body,
        out_shape=jax.ShapeDtypeStruct((B, H), jnp.float32),
    )(x2, wq3, bq2, bin_proj)

    out = pl.pallas_call(
        _flash_body,
        grid=(_NT,),
        in_specs=[
            pl.BlockSpec((B, H), lambda t: (0, 0)),
            pl.BlockSpec((_TR, B), lambda t: (t, 0)),
            pl.BlockSpec((_TR, D), lambda t: (t, 0)),
        ],
        out_specs=pl.BlockSpec((1, 1, D), lambda t: (t // _RPH, 0, 0)),
        out_shape=jax.ShapeDtypeStruct((H, 1, D), jnp.float32),
        scratch_shapes=[
            pltpu.VMEM((8, 128), jnp.float32),
            pltpu.VMEM((H, D), jnp.float32),
        ],
    )(maskt, wflat, memflat)

    return out.reshape(H * D)


# speedup vs baseline: 1.6216x; 1.3126x over previous
import jax
import jax.numpy as jnp
from jax.experimental import pallas as pl
from jax.experimental.pallas import tpu as pltpu

H = 8
D = 64
M = 8192
B = 1024
K = 64
IN = 512

_TR = 2048
_NT = (H * M) // _TR
_RPH = M // _TR

_NEG = -1e30


def _mask_body(x_ref, wq_ref, bq_ref, bp_ref, maskt_ref):
    x = x_ref[0]
    q = jnp.sum(wq_ref[:] * x[None, None, :], axis=2) + bq_ref[:]
    qn = q * jax.lax.rsqrt(jnp.sum(q * q, axis=1, keepdims=True))
    s = jnp.sum(bp_ref[:] * qn[:, None, :], axis=2)

    i = jax.lax.bitcast_convert_type(s.T, jnp.int32)
    keyt = jnp.where(i < 0, i ^ jnp.int32(0x7FFFFFFF), i)

    def count_ge(c):
        return jnp.sum((keyt >= c).astype(jnp.float32), axis=0, keepdims=True)

    def step(it, cur):
        bit = 31 - it
        cand = cur + (jnp.int32(1) << bit)
        return jnp.where(count_ge(cand) >= K, cand, cur)

    kth = jax.lax.fori_loop(0, 32, step, jnp.full((1, H), jnp.int32(-(2**31))))

    gt = (keyt > kth).astype(jnp.float32)
    n_gt = jnp.sum(gt, axis=0, keepdims=True)
    tie = (keyt == kth).astype(jnp.bfloat16)
    r = jax.lax.broadcasted_iota(jnp.int32, (B, B), 0)
    c = jax.lax.broadcasted_iota(jnp.int32, (B, B), 1)
    lt = (c < r).astype(jnp.bfloat16)
    tie_rank = jnp.dot(lt, tie, preferred_element_type=jnp.float32)
    sel_tie = tie.astype(jnp.float32) * (tie_rank < (K - n_gt)).astype(jnp.float32)
    maskt_ref[:] = gt + sel_tie


def _flash_body(maskt_ref, w_ref, mem_ref, out_ref, stat_ref, acc_ref):
    t = pl.program_id(0)
    h = t // _RPH
    oh = (jax.lax.broadcasted_iota(jnp.int32, (1, H), 1) == h).astype(jnp.float32)
    mh = (maskt_ref[:] * oh).astype(jnp.bfloat16)
    r = jnp.dot(
        w_ref[:].astype(jnp.bfloat16), mh, preferred_element_type=jnp.float32
    )
    a = r * (1.0 / K)
    sq = (2.0 * a) / (1.0 + a) * 10.0
    l = jnp.where(oh > 0.0, sq, _NEG)
    mloc = jnp.max(jnp.max(l, axis=0, keepdims=True), axis=1, keepdims=True)

    @pl.when(t % _RPH == 0)
    def _():
        stat_ref[0:1, 0:1] = jnp.full((1, 1), _NEG, jnp.float32)
        stat_ref[1:2, 0:1] = jnp.zeros((1, 1), jnp.float32)
        acc_ref[:] = jnp.zeros_like(acc_ref)

    m_old = stat_ref[0:1, 0:1]
    s_old = stat_ref[1:2, 0:1]
    m_new = jnp.maximum(m_old, mloc)
    corr = jnp.exp(m_old - m_new)
    e = jnp.exp(l - m_new)

    mem = mem_ref[:]
    inv = jax.lax.rsqrt(jnp.sum(mem * mem, axis=1, keepdims=True))
    ew = e * inv
    contrib = jax.lax.dot_general(
        ew, mem, (((0,), (0,)), ((), ())), preferred_element_type=jnp.float32
    )
    s_loc = jnp.sum(jnp.sum(e, axis=0, keepdims=True), axis=1, keepdims=True)
    stat_ref[0:1, 0:1] = m_new
    stat_ref[1:2, 0:1] = s_old * corr + s_loc
    acc_ref[:] = acc_ref[:] * corr + contrib

    @pl.when(t % _RPH == _RPH - 1)
    def _():
        s_tot = stat_ref[1:2, 0:1]
        o = jnp.sum(acc_ref[:], axis=0, keepdims=True) / s_tot * jnp.sqrt(float(D))
        out_ref[0] = o


def kernel(x, Wq, bq, bin_proj, weight_matrix, memories):
    x2 = x.reshape(1, IN)
    wq3 = Wq.reshape(H, D, IN)
    bq2 = bq.reshape(H, D)
    wflat = weight_matrix.reshape(H * M, B)
    memflat = memories.reshape(H * M, D)

    maskt = pl.pallas_call(
        _mask_body,
        out_shape=jax.ShapeDtypeStruct((B, H), jnp.float32),
    )(x2, wq3, bq2, bin_proj)

    out = pl.pallas_call(
        _flash_body,
        grid=(_NT,),
        in_specs=[
            pl.BlockSpec((B, H), lambda t: (0, 0)),
            pl.BlockSpec((_TR, B), lambda t: (t, 0)),
            pl.BlockSpec((_TR, D), lambda t: (t, 0)),
        ],
        out_specs=pl.BlockSpec((1, 1, D), lambda t: (t // _RPH, 0, 0)),
        out_shape=jax.ShapeDtypeStruct((H, 1, D), jnp.float32),
        scratch_shapes=[
            pltpu.VMEM((8, 128), jnp.float32),
            pltpu.VMEM((H, D), jnp.float32),
        ],
    )(maskt, wflat, memflat)

    return out.reshape(H * D)


# speedup vs baseline: 1.6408x; 1.0118x over previous
import jax
import jax.numpy as jnp
from jax.experimental import pallas as pl
from jax.experimental.pallas import tpu as pltpu

H = 8
D = 64
M = 8192
B = 1024
K = 64
IN = 512

_TR = 4096
_NT = (H * M) // _TR
_RPH = M // _TR

_NEG = -1e30


def _mask_body(x_ref, wq_ref, bq_ref, bp_ref, maskt_ref):
    x = x_ref[0]
    q = jnp.sum(wq_ref[:] * x[None, None, :], axis=2) + bq_ref[:]
    qn = q * jax.lax.rsqrt(jnp.sum(q * q, axis=1, keepdims=True))
    s = jnp.sum(bp_ref[:] * qn[:, None, :], axis=2)

    i = jax.lax.bitcast_convert_type(s.T, jnp.int32)
    keyt = jnp.where(i < 0, i ^ jnp.int32(0x7FFFFFFF), i)

    def count_ge(c):
        return jnp.sum((keyt >= c).astype(jnp.float32), axis=0, keepdims=True)

    def step(it, cur):
        bit = 31 - it
        cand = cur + (jnp.int32(1) << bit)
        return jnp.where(count_ge(cand) >= K, cand, cur)

    kth = jax.lax.fori_loop(0, 32, step, jnp.full((1, H), jnp.int32(-(2**31))))

    gt = (keyt > kth).astype(jnp.float32)
    n_gt = jnp.sum(gt, axis=0, keepdims=True)
    tie = (keyt == kth).astype(jnp.bfloat16)
    r = jax.lax.broadcasted_iota(jnp.int32, (B, B), 0)
    c = jax.lax.broadcasted_iota(jnp.int32, (B, B), 1)
    lt = (c < r).astype(jnp.bfloat16)
    tie_rank = jnp.dot(lt, tie, preferred_element_type=jnp.float32)
    sel_tie = tie.astype(jnp.float32) * (tie_rank < (K - n_gt)).astype(jnp.float32)
    maskt_ref[:] = gt + sel_tie


def _flash_body(maskt_ref, w_ref, mem_ref, out_ref, stat_ref, acc_ref):
    t = pl.program_id(0)
    h = t // _RPH
    oh = (jax.lax.broadcasted_iota(jnp.int32, (1, H), 1) == h).astype(jnp.float32)
    mh = (maskt_ref[:] * oh).astype(jnp.bfloat16)
    r = jnp.dot(
        w_ref[:].astype(jnp.bfloat16), mh, preferred_element_type=jnp.float32
    )
    a = r * (1.0 / K)
    sq = (2.0 * a) / (1.0 + a) * 10.0
    l = jnp.where(oh > 0.0, sq, _NEG)
    mloc = jnp.max(jnp.max(l, axis=0, keepdims=True), axis=1, keepdims=True)

    @pl.when(t % _RPH == 0)
    def _():
        stat_ref[0:1, 0:1] = jnp.full((1, 1), _NEG, jnp.float32)
        stat_ref[1:2, 0:1] = jnp.zeros((1, 1), jnp.float32)
        acc_ref[:] = jnp.zeros_like(acc_ref)

    m_old = stat_ref[0:1, 0:1]
    s_old = stat_ref[1:2, 0:1]
    m_new = jnp.maximum(m_old, mloc)
    corr = jnp.exp(m_old - m_new)
    e = jnp.exp(l - m_new)

    mem = mem_ref[:]
    inv = jax.lax.rsqrt(jnp.sum(mem * mem, axis=1, keepdims=True))
    ew = e * inv
    contrib = jax.lax.dot_general(
        ew, mem, (((0,), (0,)), ((), ())), preferred_element_type=jnp.float32
    )
    s_loc = jnp.sum(jnp.sum(e, axis=0, keepdims=True), axis=1, keepdims=True)
    stat_ref[0:1, 0:1] = m_new
    stat_ref[1:2, 0:1] = s_old * corr + s_loc
    acc_ref[:] = acc_ref[:] * corr + contrib

    @pl.when(t % _RPH == _RPH - 1)
    def _():
        s_tot = stat_ref[1:2, 0:1]
        o = jnp.sum(acc_ref[:], axis=0, keepdims=True) / s_tot * jnp.sqrt(float(D))
        out_ref[0] = o


def kernel(x, Wq, bq, bin_proj, weight_matrix, memories):
    x2 = x.reshape(1, IN)
    wq3 = Wq.reshape(H, D, IN)
    bq2 = bq.reshape(H, D)
    wflat = weight_matrix.reshape(H * M, B)
    memflat = memories.reshape(H * M, D)

    maskt = pl.pallas_call(
        _mask_body,
        out_shape=jax.ShapeDtypeStruct((B, H), jnp.float32),
    )(x2, wq3, bq2, bin_proj)

    out = pl.pallas_call(
        _flash_body,
        grid=(_NT,),
        in_specs=[
            pl.BlockSpec((B, H), lambda t: (0, 0)),
            pl.BlockSpec((_TR, B), lambda t: (t, 0)),
            pl.BlockSpec((_TR, D), lambda t: (t, 0)),
        ],
        out_specs=pl.BlockSpec((1, 1, D), lambda t: (t // _RPH, 0, 0)),
        out_shape=jax.ShapeDtypeStruct((H, 1, D), jnp.float32),
        scratch_shapes=[
            pltpu.VMEM((8, 128), jnp.float32),
            pltpu.VMEM((H, D), jnp.float32),
        ],
    )(maskt, wflat, memflat)

    return out.reshape(H * D)
